# Initial kernel scaffold; baseline (speedup 1.0000x reference)
import jax
import jax.numpy as jnp
from jax.experimental import pallas as pl


def kernel(objectness, pred_bbox_delta, anchor, image_shape):
    raise NotImplementedError("write your pallas kernel here")


# speedup vs baseline: 8.2906x; 8.2906x over previous
import jax
import jax.numpy as jnp
from jax.experimental import pallas as pl
from jax.experimental.pallas import tpu as pltpu

_NPRE = 2000
_K = 2048
_BLK = 128
_NBLK = _K // _BLK
_CHUNK = 256
_POST = 300
_OUTW = 384
_TH = 0.7


def _tcol(v):
    i = jax.lax.broadcasted_iota(jnp.int32, (_BLK, _BLK), 0)
    j = jax.lax.broadcasted_iota(jnp.int32, (_BLK, _BLK), 1)
    diag = (i == j).astype(jnp.float32)
    vb = jnp.broadcast_to(v, (_BLK, _BLK))
    return jnp.sum(diag * vb, axis=1, keepdims=True)


def _tcol_full(v):
    return jnp.concatenate(
        [_tcol(v[:, c * _BLK:(c + 1) * _BLK]) for c in range(_NBLK)], axis=0)


def _cumsum_row(v):
    ki = jax.lax.broadcasted_iota(jnp.int32, (_BLK, _BLK), 0)
    ji = jax.lax.broadcasted_iota(jnp.int32, (_BLK, _BLK), 1)
    lt = (ki <= ji).astype(jnp.float32)
    carry = jnp.zeros((1, 1), jnp.float32)
    outs = []
    for c in range(_NBLK):
        w = jnp.dot(v[:, c * _BLK:(c + 1) * _BLK], lt,
                    preferred_element_type=jnp.float32)
        outs.append(w + carry)
        carry = carry + w[:, _BLK - 1:_BLK]
    return jnp.concatenate(outs, axis=1)


def _compact_dest(flag_f, flag_b):
    cf = _cumsum_row(flag_f)
    cn = _cumsum_row(1.0 - flag_f)
    nf = cf[:, _K - 1:_K]
    return jnp.where(flag_b, cf - 1.0, nf + cn - 1.0)


def _proposal_body(in_ref, out_ref, s_ref, k_ref):
    x = in_ref[0]
    dl = x[0:4]
    an = x[4:8]
    h = x[8:9, 0:1]
    w = x[9:10, 0:1]

    ax1, ay1, ax2, ay2 = an[0:1], an[1:2], an[2:3], an[3:4]
    aw = ax2 - ax1
    ah = ay2 - ay1
    acx = ax1 + 0.5 * aw
    acy = ay1 + 0.5 * ah
    dx, dy = dl[0:1], dl[1:2]
    dw = jnp.minimum(dl[2:3], 4.0)
    dh = jnp.minimum(dl[3:4], 4.0)
    cx = dx * aw + acx
    cy = dy * ah + acy
    bw = jnp.exp(dw) * aw
    bh = jnp.exp(dh) * ah
    x1 = jnp.clip(cx - 0.5 * bw, 0.0, w)
    y1 = jnp.clip(cy - 0.5 * bh, 0.0, h)
    x2 = jnp.clip(cx + 0.5 * bw, 0.0, w)
    y2 = jnp.clip(cy + 0.5 * bh, 0.0, h)
    valid = ((x2 - x1) >= 1.0) & ((y2 - y1) >= 1.0)
    vf = valid.astype(jnp.float32)

    dest1 = _compact_dest(vf, valid)
    d1c = _tcol_full(dest1)
    src = jnp.concatenate(
        [x1, y1, x2, y2, vf, jnp.zeros((3, _K), jnp.float32)], axis=0)
    sorted_chunks = []
    for c in range(_K // _CHUNK):
        jv = (jax.lax.broadcasted_iota(jnp.int32, (1, _CHUNK), 1)
              + c * _CHUNK).astype(jnp.float32)
        perm = (d1c == jv).astype(jnp.float32)
        sorted_chunks.append(
            jnp.dot(src, perm, preferred_element_type=jnp.float32,
                    precision=jax.lax.Precision.HIGHEST))
    sm = jnp.concatenate(sorted_chunks, axis=1)
    x1s, y1s, x2s, y2s, keep = sm[0:1], sm[1:2], sm[2:3], sm[3:4], sm[4:5]
    area = jnp.maximum(x2s - x1s, 0.0) * jnp.maximum(y2s - y1s, 0.0)

    lane_b = jax.lax.broadcasted_iota(jnp.int32, (1, _BLK), 1)
    keep_segs = [keep[:, c * _BLK:(c + 1) * _BLK] for c in range(_NBLK)]
    for b in range(_NBLK):
        sl = slice(b * _BLK, (b + 1) * _BLK)
        rx1, ry1 = _tcol(x1s[:, sl]), _tcol(y1s[:, sl])
        rx2, ry2 = _tcol(x2s[:, sl]), _tcol(y2s[:, sl])
        rarea = _tcol(area[:, sl])
        iw = jnp.maximum(jnp.minimum(rx2, x2s) - jnp.maximum(rx1, x1s), 0.0)
        ih = jnp.maximum(jnp.minimum(ry2, y2s) - jnp.maximum(ry1, y1s), 0.0)
        inter = iw * ih
        union = jnp.maximum(rarea + area - inter, 1e-9)
        s = (inter / union > _TH).astype(jnp.float32)
        s_ref[...] = s[:, sl]
        k_ref[0:1, :] = keep_segs[b]

        def body(i, carry):
            kb = k_ref[0:1, :]
            srow = s_ref[pl.ds(i, 1), :]
            kbi = jnp.sum(kb * (lane_b == i).astype(jnp.float32),
                          axis=1, keepdims=True)
            mask = (lane_b > i).astype(jnp.float32)
            k_ref[0:1, :] = kb * (1.0 - srow * mask * kbi)
            return carry

        jax.lax.fori_loop(0, _BLK, body, 0)
        kb = k_ref[0:1, :]
        keep_segs[b] = kb
        sup = jnp.dot(kb, s, preferred_element_type=jnp.float32)
        for c in range(b + 1, _NBLK):
            supc = (sup[:, c * _BLK:(c + 1) * _BLK] > 0.0).astype(jnp.float32)
            keep_segs[c] = keep_segs[c] * (1.0 - supc)
    keep = jnp.concatenate(keep_segs, axis=1)

    dest2 = _compact_dest(keep, keep > 0.5)
    d2c = _tcol_full(dest2)
    jv = jax.lax.broadcasted_iota(jnp.int32, (1, _OUTW), 1).astype(jnp.float32)
    oh = (d2c == jv).astype(jnp.float32)
    kept_coords = jnp.concatenate(
        [x1s * keep, y1s * keep, x2s * keep, y2s * keep], axis=0)
    prop_t = jnp.dot(kept_coords, oh, preferred_element_type=jnp.float32,
                     precision=jax.lax.Precision.HIGHEST)
    srcpos = jax.lax.broadcasted_iota(jnp.int32, (1, _K), 1).astype(jnp.float32)
    selrow = jnp.dot(srcpos, oh, preferred_element_type=jnp.float32,
                     precision=jax.lax.Precision.HIGHEST)
    out_ref[0] = jnp.concatenate(
        [prop_t, selrow, jnp.zeros((3, _OUTW), jnp.float32)], axis=0)


def kernel(objectness, pred_bbox_delta, anchor, image_shape):
    bsz = objectness.shape[0]
    _, idx = jax.lax.top_k(objectness, _NPRE)
    d = jnp.take_along_axis(pred_bbox_delta, idx[:, :, None], axis=1)
    a = jnp.take_along_axis(anchor, idx[:, :, None], axis=1)
    pad = _K - _NPRE
    dt = jnp.pad(d, ((0, 0), (0, pad), (0, 0))).transpose(0, 2, 1)
    at = jnp.pad(a, ((0, 0), (0, pad), (0, 0))).transpose(0, 2, 1)
    hrow = jnp.broadcast_to(image_shape[0].astype(jnp.float32), (bsz, 1, _K))
    wrow = jnp.broadcast_to(image_shape[1].astype(jnp.float32), (bsz, 1, _K))
    packed = jnp.concatenate(
        [dt, at, hrow, wrow, jnp.zeros((bsz, 6, _K), jnp.float32)], axis=1)
    out = pl.pallas_call(
        _proposal_body,
        grid=(bsz,),
        in_specs=[pl.BlockSpec((1, 16, _K), lambda b: (b, 0, 0))],
        out_specs=pl.BlockSpec((1, 8, _OUTW), lambda b: (b, 0, 0)),
        out_shape=jax.ShapeDtypeStruct((bsz, 8, _OUTW), jnp.float32),
        scratch_shapes=[pltpu.VMEM((_BLK, _BLK), jnp.float32),
                        pltpu.VMEM((8, _BLK), jnp.float32)],
    )(packed)
    prop = out[:, 0:4, :_POST].transpose(0, 2, 1)
    sel = out[:, 4, :_POST].astype(jnp.int32)
    return prop, sel


# speedup vs baseline: 8.3882x; 1.0118x over previous
import jax
import jax.numpy as jnp
from jax.experimental import pallas as pl
from jax.experimental.pallas import tpu as pltpu

_NPRE = 2000
_K = 2048
_BLK = 128
_NBLK = _K // _BLK
_CHUNK = 256
_POST = 300
_OUTW = 384
_TH = 0.7


def _tcol(v):
    i = jax.lax.broadcasted_iota(jnp.int32, (_BLK, _BLK), 0)
    j = jax.lax.broadcasted_iota(jnp.int32, (_BLK, _BLK), 1)
    diag = (i == j).astype(jnp.float32)
    vb = jnp.broadcast_to(v, (_BLK, _BLK))
    return jnp.sum(diag * vb, axis=1, keepdims=True)


def _tcol_full(v):
    return jnp.concatenate(
        [_tcol(v[:, c * _BLK:(c + 1) * _BLK]) for c in range(_NBLK)], axis=0)


def _cumsum_row(v):
    ki = jax.lax.broadcasted_iota(jnp.int32, (_BLK, _BLK), 0)
    ji = jax.lax.broadcasted_iota(jnp.int32, (_BLK, _BLK), 1)
    lt = (ki <= ji).astype(jnp.float32)
    carry = jnp.zeros((1, 1), jnp.float32)
    outs = []
    for c in range(_NBLK):
        w = jnp.dot(v[:, c * _BLK:(c + 1) * _BLK], lt,
                    preferred_element_type=jnp.float32)
        outs.append(w + carry)
        carry = carry + w[:, _BLK - 1:_BLK]
    return jnp.concatenate(outs, axis=1)


def _compact_dest(flag_f, flag_b):
    cf = _cumsum_row(flag_f)
    cn = _cumsum_row(1.0 - flag_f)
    nf = cf[:, _K - 1:_K]
    return jnp.where(flag_b, cf - 1.0, nf + cn - 1.0)


def _proposal_body(in_ref, out_ref, s_ref, k_ref):
    x = in_ref[0]
    dl = x[0:4]
    an = x[4:8]
    h = x[8:9, 0:1]
    w = x[9:10, 0:1]

    ax1, ay1, ax2, ay2 = an[0:1], an[1:2], an[2:3], an[3:4]
    aw = ax2 - ax1
    ah = ay2 - ay1
    acx = ax1 + 0.5 * aw
    acy = ay1 + 0.5 * ah
    dx, dy = dl[0:1], dl[1:2]
    dw = jnp.minimum(dl[2:3], 4.0)
    dh = jnp.minimum(dl[3:4], 4.0)
    cx = dx * aw + acx
    cy = dy * ah + acy
    bw = jnp.exp(dw) * aw
    bh = jnp.exp(dh) * ah
    x1 = jnp.clip(cx - 0.5 * bw, 0.0, w)
    y1 = jnp.clip(cy - 0.5 * bh, 0.0, h)
    x2 = jnp.clip(cx + 0.5 * bw, 0.0, w)
    y2 = jnp.clip(cy + 0.5 * bh, 0.0, h)
    valid = ((x2 - x1) >= 1.0) & ((y2 - y1) >= 1.0)
    vf = valid.astype(jnp.float32)

    dest1 = _compact_dest(vf, valid)
    d1c = _tcol_full(dest1)
    src = jnp.concatenate(
        [x1, y1, x2, y2, vf, jnp.zeros((3, _K), jnp.float32)], axis=0)
    sorted_chunks = []
    for c in range(_K // _CHUNK):
        jv = (jax.lax.broadcasted_iota(jnp.int32, (1, _CHUNK), 1)
              + c * _CHUNK).astype(jnp.float32)
        perm = (d1c == jv).astype(jnp.float32)
        sorted_chunks.append(
            jnp.dot(src, perm, preferred_element_type=jnp.float32,
                    precision=jax.lax.Precision.HIGHEST))
    sm = jnp.concatenate(sorted_chunks, axis=1)
    x1s, y1s, x2s, y2s, keep = sm[0:1], sm[1:2], sm[2:3], sm[3:4], sm[4:5]
    area = jnp.maximum(x2s - x1s, 0.0) * jnp.maximum(y2s - y1s, 0.0)

    lane_b = jax.lax.broadcasted_iota(jnp.int32, (1, _BLK), 1)
    keep_segs = [keep[:, c * _BLK:(c + 1) * _BLK] for c in range(_NBLK)]
    for b in range(_NBLK):
        sl = slice(b * _BLK, (b + 1) * _BLK)
        rx1, ry1 = _tcol(x1s[:, sl]), _tcol(y1s[:, sl])
        rx2, ry2 = _tcol(x2s[:, sl]), _tcol(y2s[:, sl])
        rarea = _tcol(area[:, sl])
        iw = jnp.maximum(jnp.minimum(rx2, x2s) - jnp.maximum(rx1, x1s), 0.0)
        ih = jnp.maximum(jnp.minimum(ry2, y2s) - jnp.maximum(ry1, y1s), 0.0)
        inter = iw * ih
        union = jnp.maximum(rarea + area - inter, 1e-9)
        s = (inter / union > _TH).astype(jnp.float32)
        s_ref[...] = s[:, sl]
        k_ref[0:1, :] = keep_segs[b]

        def body(i, carry):
            kb = k_ref[0:1, :]
            srow = s_ref[pl.ds(i, 1), :]
            kbi = jnp.sum(kb * (lane_b == i).astype(jnp.float32),
                          axis=1, keepdims=True)
            mask = (lane_b > i).astype(jnp.float32)
            k_ref[0:1, :] = kb * (1.0 - srow * mask * kbi)
            return carry

        jax.lax.fori_loop(0, _BLK, body, 0, unroll=8)
        kb = k_ref[0:1, :]
        keep_segs[b] = kb
        sup = jnp.dot(kb, s, preferred_element_type=jnp.float32)
        for c in range(b + 1, _NBLK):
            supc = (sup[:, c * _BLK:(c + 1) * _BLK] > 0.0).astype(jnp.float32)
            keep_segs[c] = keep_segs[c] * (1.0 - supc)
    keep = jnp.concatenate(keep_segs, axis=1)

    dest2 = _compact_dest(keep, keep > 0.5)
    d2c = _tcol_full(dest2)
    jv = jax.lax.broadcasted_iota(jnp.int32, (1, _OUTW), 1).astype(jnp.float32)
    oh = (d2c == jv).astype(jnp.float32)
    kept_coords = jnp.concatenate(
        [x1s * keep, y1s * keep, x2s * keep, y2s * keep], axis=0)
    prop_t = jnp.dot(kept_coords, oh, preferred_element_type=jnp.float32,
                     precision=jax.lax.Precision.HIGHEST)
    srcpos = jax.lax.broadcasted_iota(jnp.int32, (1, _K), 1).astype(jnp.float32)
    selrow = jnp.dot(srcpos, oh, preferred_element_type=jnp.float32,
                     precision=jax.lax.Precision.HIGHEST)
    out_ref[0] = jnp.concatenate(
        [prop_t, selrow, jnp.zeros((3, _OUTW), jnp.float32)], axis=0)


def kernel(objectness, pred_bbox_delta, anchor, image_shape):
    bsz = objectness.shape[0]
    _, idx = jax.lax.top_k(objectness, _NPRE)
    d = jnp.take_along_axis(pred_bbox_delta, idx[:, :, None], axis=1)
    a = jnp.take_along_axis(anchor, idx[:, :, None], axis=1)
    pad = _K - _NPRE
    dt = jnp.pad(d, ((0, 0), (0, pad), (0, 0))).transpose(0, 2, 1)
    at = jnp.pad(a, ((0, 0), (0, pad), (0, 0))).transpose(0, 2, 1)
    hrow = jnp.broadcast_to(image_shape[0].astype(jnp.float32), (bsz, 1, _K))
    wrow = jnp.broadcast_to(image_shape[1].astype(jnp.float32), (bsz, 1, _K))
    packed = jnp.concatenate(
        [dt, at, hrow, wrow, jnp.zeros((bsz, 6, _K), jnp.float32)], axis=1)
    out = pl.pallas_call(
        _proposal_body,
        grid=(bsz,),
        in_specs=[pl.BlockSpec((1, 16, _K), lambda b: (b, 0, 0))],
        out_specs=pl.BlockSpec((1, 8, _OUTW), lambda b: (b, 0, 0)),
        out_shape=jax.ShapeDtypeStruct((bsz, 8, _OUTW), jnp.float32),
        scratch_shapes=[pltpu.VMEM((_BLK, _BLK), jnp.float32),
                        pltpu.VMEM((8, _BLK), jnp.float32)],
    )(packed)
    prop = out[:, 0:4, :_POST].transpose(0, 2, 1)
    sel = out[:, 4, :_POST].astype(jnp.int32)
    return prop, sel
